# baseline (speedup 1.0000x reference)
import functools

import jax
import jax.numpy as jnp
from jax import lax
from jax.experimental import pallas as pl
from jax.experimental.pallas import tpu as pltpu
from jax.experimental.pallas import tpu_sc as plsc

N_TRIPLES = 160000
EMB_DIM = 256
HALF = EMB_DIM // 2
LANES = 16
NUM_CORES = 2
NUM_SUBCORES = 16
NW = NUM_CORES * NUM_SUBCORES
PER_W = N_TRIPLES // NW
TRI_W = PER_W * 3

CHUNK = 128
NCHUNK_W = 40
LAST_START = PER_W - CHUNK
BREGS = EMB_DIM // (2 * LANES)
GROUPS = CHUNK // LANES

HI_MASK = -65536

PACK_ROWS = 1000


def _pack_one(x):
    lo = lax.bitcast_convert_type(
        x[:, :HALF].astype(jnp.bfloat16), jnp.uint16).astype(jnp.uint32)
    hi = lax.bitcast_convert_type(
        x[:, HALF:].astype(jnp.bfloat16), jnp.uint16).astype(jnp.uint32)
    return lax.bitcast_convert_type(lo | (hi << 16), jnp.int32)


def _pack_body(e_ref, r_ref, eo_ref, ro_ref):
    eo_ref[...] = _pack_one(e_ref[...])
    ro_ref[...] = _pack_one(r_ref[...])


def _pack_tables(ent, rel):
    n = ent.shape[0]
    return pl.pallas_call(
        _pack_body,
        grid=(n // PACK_ROWS,),
        in_specs=[
            pl.BlockSpec((PACK_ROWS, EMB_DIM), lambda i: (i, 0)),
            pl.BlockSpec((PACK_ROWS, EMB_DIM), lambda i: (i, 0)),
        ],
        out_specs=[
            pl.BlockSpec((PACK_ROWS, HALF), lambda i: (i, 0)),
            pl.BlockSpec((PACK_ROWS, HALF), lambda i: (i, 0)),
        ],
        out_shape=(
            jax.ShapeDtypeStruct((n, HALF), jnp.int32),
            jax.ShapeDtypeStruct((rel.shape[0], HALF), jnp.int32),
        ),
    )(ent, rel)


def _sc_body(tri_hbm, ent_hbm, rel_hbm, wb_hbm, out_hbm,
             tri, hixc, rixc, tixc, hrows, rrows, trows, ob, wbv,
             sem0, sem1):
    wid = lax.axis_index("s") * NUM_CORES + lax.axis_index("c")
    base_w = pl.multiple_of(wid * PER_W, 8)

    pltpu.sync_copy(wb_hbm, wbv)
    wv = wbv[0, :]
    bv = wbv[1, :]
    pltpu.sync_copy(tri_hbm.at[pl.ds(pl.multiple_of(wid * TRI_W, 8), TRI_W)],
                    tri)

    sems = (sem0, sem1)
    lane_iota = lax.iota(jnp.int32, LANES)
    lane3 = lane_iota * 3

    def chunk_start(c):
        return pl.multiple_of(jnp.minimum(c * CHUNK, LAST_START), 8)

    def issue(c, ph):
        start = chunk_start(c)
        for g in range(GROUPS):
            gbase = (start + g * LANES) * 3
            sl = pl.ds(g * LANES, LANES)
            hixc[ph, sl] = plsc.load_gather(tri, [lane3 + gbase])
            rixc[ph, sl] = plsc.load_gather(tri, [lane3 + (gbase + 1)])
            tixc[ph, sl] = plsc.load_gather(tri, [lane3 + (gbase + 2)])
        pltpu.async_copy(ent_hbm.at[hixc.at[ph]], hrows.at[ph], sems[ph])
        pltpu.async_copy(rel_hbm.at[rixc.at[ph]], rrows.at[ph], sems[ph])
        pltpu.async_copy(ent_hbm.at[tixc.at[ph]], trows.at[ph], sems[ph])

    def wait_rows(ph):
        pltpu.make_async_copy(ent_hbm.at[hixc.at[ph]],
                              hrows.at[ph], sems[ph]).wait()
        pltpu.make_async_copy(rel_hbm.at[rixc.at[ph]],
                              rrows.at[ph], sems[ph]).wait()
        pltpu.make_async_copy(ent_hbm.at[tixc.at[ph]],
                              trows.at[ph], sems[ph]).wait()

    def compute(c, ph):
        start = chunk_start(c)
        for grp in range(GROUPS):
            def triple_body(j, score):
                row = grp * LANES + j
                acc0 = None
                acc1 = None
                for k in range(BREGS):
                    sl = pl.ds(k * LANES, LANES)
                    hp = plsc.bitcast(hrows[ph, row, sl], jnp.bfloat16)
                    rp = plsc.bitcast(rrows[ph, row, sl], jnp.bfloat16)
                    tp = plsc.bitcast(trows[ph, row, sl], jnp.bfloat16)
                    pi = plsc.bitcast(hp * rp * tp, jnp.int32)
                    lo = plsc.bitcast(pi << 16, jnp.float32)
                    hi = plsc.bitcast(pi & HI_MASK, jnp.float32)
                    acc0 = lo if acc0 is None else acc0 + lo
                    acc1 = hi if acc1 is None else acc1 + hi
                s = jnp.sum(acc0 + acc1)
                return jnp.where(lane_iota == j, s, score)

            score = lax.fori_loop(
                0, LANES, triple_body,
                jnp.zeros((LANES,), jnp.float32), unroll=4)
            score = 1.0 / (1.0 + jnp.exp(-(wv * score + bv)))
            ob[pl.ds(start + grp * LANES, LANES)] = score

    issue(0, 0)

    def pair_body(i2, _):
        a = 2 * i2
        issue(a + 1, 1)
        wait_rows(0)
        compute(a, 0)

        @pl.when(a + 2 < NCHUNK_W)
        def _():
            issue(a + 2, 0)

        wait_rows(1)
        compute(a + 1, 1)
        return 0

    lax.fori_loop(0, NCHUNK_W // 2, pair_body, 0)

    pltpu.sync_copy(ob, out_hbm.at[pl.ds(base_w, PER_W)])


def kernel(triples, ent_emb, rel_emb, w, b):
    tri_flat = triples.astype(jnp.int32).reshape(-1)
    ent_bf, rel_bf = _pack_tables(ent_emb, rel_emb)
    wb = jnp.stack([
        jnp.full((LANES,), w, jnp.float32),
        jnp.full((LANES,), b, jnp.float32),
    ])

    mesh = plsc.VectorSubcoreMesh(
        core_axis_name="c", subcore_axis_name="s",
        num_cores=NUM_CORES, num_subcores=NUM_SUBCORES)

    sc_call = functools.partial(
        pl.kernel,
        mesh=mesh,
        compiler_params=pltpu.CompilerParams(needs_layout_passes=False),
        out_type=jax.ShapeDtypeStruct((N_TRIPLES,), jnp.float32),
        scratch_types=[
            pltpu.VMEM((TRI_W,), jnp.int32),
            pltpu.VMEM((2, CHUNK), jnp.int32),
            pltpu.VMEM((2, CHUNK), jnp.int32),
            pltpu.VMEM((2, CHUNK), jnp.int32),
            pltpu.VMEM((2, CHUNK, HALF), jnp.int32),
            pltpu.VMEM((2, CHUNK, HALF), jnp.int32),
            pltpu.VMEM((2, CHUNK, HALF), jnp.int32),
            pltpu.VMEM((PER_W,), jnp.float32),
            pltpu.VMEM((2, LANES), jnp.float32),
            pltpu.SemaphoreType.DMA,
            pltpu.SemaphoreType.DMA,
        ],
    )(_sc_body)

    return sc_call(tri_flat, ent_bf, rel_bf, wb)


# speedup vs baseline: 1.9250x; 1.9250x over previous
import functools

import jax
import jax.numpy as jnp
from jax import lax
from jax.experimental import pallas as pl
from jax.experimental.pallas import tpu as pltpu
from jax.experimental.pallas import tpu_sc as plsc

N_TRIPLES = 160000
EMB_DIM = 256
HALF = EMB_DIM // 2
LANES = 16
NUM_CORES = 2
NUM_SUBCORES = 16
NW = NUM_CORES * NUM_SUBCORES
PER_W = N_TRIPLES // NW
TRI_W = PER_W * 3

CHUNK = 128
NCHUNK_W = 40
LAST_START = PER_W - CHUNK
BREGS = EMB_DIM // (2 * LANES)
GROUPS = CHUNK // LANES

HI_MASK = -65536

PACK_ROWS = 1000


def _pack_one(x):
    lo = lax.bitcast_convert_type(
        x[:, :HALF].astype(jnp.bfloat16), jnp.uint16).astype(jnp.uint32)
    hi = lax.bitcast_convert_type(
        x[:, HALF:].astype(jnp.bfloat16), jnp.uint16).astype(jnp.uint32)
    return lax.bitcast_convert_type(lo | (hi << 16), jnp.int32)


def _pack_body(e_ref, r_ref, eo_ref, ro_ref):
    eo_ref[...] = _pack_one(e_ref[...])
    ro_ref[...] = _pack_one(r_ref[...])


def _pack_tables(ent, rel):
    return pl.pallas_call(
        _pack_body,
        out_shape=(
            jax.ShapeDtypeStruct((ent.shape[0], HALF), jnp.int32),
            jax.ShapeDtypeStruct((rel.shape[0], HALF), jnp.int32),
        ),
    )(ent, rel)


def _sc_body(hidx_hbm, ridx_hbm, tidx_hbm, ent_hbm, rel_hbm, wb_hbm, out_hbm,
             hix, rix, tix, hrows, rrows, trows, ob, wbv, sem0, sem1):
    wid = lax.axis_index("s") * NUM_CORES + lax.axis_index("c")
    base_w = pl.multiple_of(wid * PER_W, 8)

    pltpu.sync_copy(wb_hbm, wbv)
    wv = wbv[0, :]
    bv = wbv[1, :]
    pltpu.sync_copy(hidx_hbm.at[pl.ds(base_w, PER_W)], hix)
    pltpu.sync_copy(ridx_hbm.at[pl.ds(base_w, PER_W)], rix)
    pltpu.sync_copy(tidx_hbm.at[pl.ds(base_w, PER_W)], tix)

    sems = (sem0, sem1)
    lane_iota = lax.iota(jnp.int32, LANES)

    def chunk_start(c):
        return pl.multiple_of(jnp.minimum(c * CHUNK, LAST_START), 8)

    def issue(c, ph):
        start = chunk_start(c)
        pltpu.async_copy(ent_hbm.at[hix.at[pl.ds(start, CHUNK)]],
                         hrows.at[ph], sems[ph])
        pltpu.async_copy(rel_hbm.at[rix.at[pl.ds(start, CHUNK)]],
                         rrows.at[ph], sems[ph])
        pltpu.async_copy(ent_hbm.at[tix.at[pl.ds(start, CHUNK)]],
                         trows.at[ph], sems[ph])

    def wait_rows(ph):
        pltpu.make_async_copy(ent_hbm.at[hix.at[pl.ds(0, CHUNK)]],
                              hrows.at[ph], sems[ph]).wait()
        pltpu.make_async_copy(rel_hbm.at[rix.at[pl.ds(0, CHUNK)]],
                              rrows.at[ph], sems[ph]).wait()
        pltpu.make_async_copy(ent_hbm.at[tix.at[pl.ds(0, CHUNK)]],
                              trows.at[ph], sems[ph]).wait()

    def compute(c, ph):
        start = chunk_start(c)
        for grp in range(GROUPS):
            def triple_body(j, score):
                row = grp * LANES + j
                acc0 = None
                acc1 = None
                for k in range(BREGS):
                    sl = pl.ds(k * LANES, LANES)
                    hp = plsc.bitcast(hrows[ph, row, sl], jnp.bfloat16)
                    rp = plsc.bitcast(rrows[ph, row, sl], jnp.bfloat16)
                    tp = plsc.bitcast(trows[ph, row, sl], jnp.bfloat16)
                    pi = plsc.bitcast(hp * rp * tp, jnp.int32)
                    lo = plsc.bitcast(pi << 16, jnp.float32)
                    hi = plsc.bitcast(pi & HI_MASK, jnp.float32)
                    acc0 = lo if acc0 is None else acc0 + lo
                    acc1 = hi if acc1 is None else acc1 + hi
                s = jnp.sum(acc0 + acc1)
                return jnp.where(lane_iota == j, s, score)

            score = lax.fori_loop(
                0, LANES, triple_body,
                jnp.zeros((LANES,), jnp.float32), unroll=2)
            score = 1.0 / (1.0 + jnp.exp(-(wv * score + bv)))
            ob[pl.ds(start + grp * LANES, LANES)] = score

    issue(0, 0)

    def pair_body(i2, _):
        a = 2 * i2
        issue(a + 1, 1)
        wait_rows(0)
        compute(a, 0)

        @pl.when(a + 2 < NCHUNK_W)
        def _():
            issue(a + 2, 0)

        wait_rows(1)
        compute(a + 1, 1)
        return 0

    lax.fori_loop(0, NCHUNK_W // 2, pair_body, 0)

    pltpu.sync_copy(ob, out_hbm.at[pl.ds(base_w, PER_W)])


def kernel(triples, ent_emb, rel_emb, w, b):
    h_idx = triples[:, 0].astype(jnp.int32)
    r_idx = triples[:, 1].astype(jnp.int32)
    t_idx = triples[:, 2].astype(jnp.int32)
    ent_bf, rel_bf = _pack_tables(ent_emb, rel_emb)
    wb = jnp.stack([
        jnp.full((LANES,), w, jnp.float32),
        jnp.full((LANES,), b, jnp.float32),
    ])

    mesh = plsc.VectorSubcoreMesh(
        core_axis_name="c", subcore_axis_name="s",
        num_cores=NUM_CORES, num_subcores=NUM_SUBCORES)

    sc_call = functools.partial(
        pl.kernel,
        mesh=mesh,
        compiler_params=pltpu.CompilerParams(needs_layout_passes=False),
        out_type=jax.ShapeDtypeStruct((N_TRIPLES,), jnp.float32),
        scratch_types=[
            pltpu.VMEM((PER_W,), jnp.int32),
            pltpu.VMEM((PER_W,), jnp.int32),
            pltpu.VMEM((PER_W,), jnp.int32),
            pltpu.VMEM((2, CHUNK, HALF), jnp.int32),
            pltpu.VMEM((2, CHUNK, HALF), jnp.int32),
            pltpu.VMEM((2, CHUNK, HALF), jnp.int32),
            pltpu.VMEM((PER_W,), jnp.float32),
            pltpu.VMEM((2, LANES), jnp.float32),
            pltpu.SemaphoreType.DMA,
            pltpu.SemaphoreType.DMA,
        ],
    )(_sc_body)

    return sc_call(h_idx, r_idx, t_idx, ent_bf, rel_bf, wb)
